# baseline (speedup 1.0000x reference)
import jax
import jax.numpy as jnp
from jax.experimental import pallas as pl
from jax.experimental.pallas import tpu as pltpu

L_IN = 420
KW = 28
L_OUT = 15
C_OUT = 32
HID = 32
F = L_OUT * C_OUT
P0 = 8
P1 = L_OUT - P0
K0 = P0 * KW
K1 = P1 * KW
N0 = P0 * C_OUT
N1 = P1 * C_OUT
NP = 256


def _net_kernel(x_ref, w0_ref, bb0_ref, w1a_ref,
                w1c_ref, bb1_ref, w1b_ref,
                b1_ref, w2_ref, b2_ref, w3_ref, b3_ref, out_ref):
    xb = x_ref[...]
    zero = jnp.bfloat16(0.0)
    h0 = jnp.dot(w0_ref[...], xb[:K0, :],
                 preferred_element_type=jnp.float32).astype(jnp.bfloat16)
    h0 = jnp.maximum(h0 + bb0_ref[...], zero)
    h1 = jnp.dot(w1c_ref[...], xb[K0:, :],
                 preferred_element_type=jnp.float32).astype(jnp.bfloat16)
    h1 = jnp.maximum(h1 + bb1_ref[...], zero)
    yt = (jnp.dot(w1a_ref[...], h0, preferred_element_type=jnp.float32)
          + jnp.dot(w1b_ref[...], h1, preferred_element_type=jnp.float32))
    yt = jnp.maximum(yt.astype(jnp.bfloat16) + b1_ref[...], zero)
    zt = jnp.dot(w2_ref[...], yt, preferred_element_type=jnp.float32)
    zt = jnp.maximum(zt.astype(jnp.bfloat16) + b2_ref[...], zero)
    logit = jnp.dot(w3_ref[...], zt,
                    preferred_element_type=jnp.float32) + b3_ref[...]
    out_ref[...] = jax.nn.sigmoid(logit)


def _prep_weights(wc, bc, w1, b1, w2, b2, w3, b3):
    wct = jnp.transpose(wc[:, 0, :]).astype(jnp.float32)
    def blockdiag_t(npos):
        eye = jnp.eye(npos, dtype=jnp.float32)
        w = jnp.einsum('lm,kc->lkmc', eye, wct).reshape(npos * KW, npos * C_OUT)
        return jnp.transpose(w)

    w0 = blockdiag_t(P0)
    w1c = jnp.pad(blockdiag_t(P1), ((0, NP - N1), (0, 0)))
    bb0 = jnp.tile(bc, P0).reshape(N0, 1).astype(jnp.bfloat16)
    bb1 = jnp.pad(jnp.tile(bc, P1), (0, NP - N1)).reshape(NP, 1).astype(jnp.bfloat16)
    w1r = jnp.transpose(w1.reshape(HID, C_OUT, L_OUT), (2, 1, 0)).reshape(F, HID)
    w1t = jnp.transpose(w1r)
    w1a = w1t[:, :N0].astype(jnp.bfloat16)
    w1b = jnp.pad(w1t[:, N0:], ((0, 0), (0, NP - N1))).astype(jnp.bfloat16)
    b1c = b1.reshape(HID, 1).astype(jnp.bfloat16)
    w2n = w2.astype(jnp.bfloat16)
    b2c = b2.reshape(HID, 1).astype(jnp.bfloat16)
    w3n = w3.astype(jnp.bfloat16)
    b3r = b3.reshape(1, 1)
    return w0, bb0, w1a, w1c, bb1, w1b, b1c, w2n, b2c, w3n, b3r


@jax.jit
def kernel(x, wc, bc, w1, b1, w2, b2, w3, b3):
    B = x.shape[0]
    weights = _prep_weights(wc, bc, w1, b1, w2, b2, w3, b3)

    x_t = jnp.transpose(x.reshape(B, L_IN))

    TB = min(2048, max(128, ((B + 127) // 128) * 128))
    Bp = ((B + TB - 1) // TB) * TB
    if Bp != B:
        x_t = jnp.pad(x_t, ((0, 0), (0, Bp - B)))
    grid = (Bp // TB,)

    def wspec(shape):
        return pl.BlockSpec(shape, lambda i: (0, 0))

    out = pl.pallas_call(
        _net_kernel,
        out_shape=jax.ShapeDtypeStruct((1, Bp), jnp.float32),
        grid=grid,
        in_specs=[pl.BlockSpec((L_IN, TB), lambda i: (0, i)),
                  wspec((N0, K0)), wspec((N0, 1)), wspec((HID, N0)),
                  wspec((NP, K1)), wspec((NP, 1)), wspec((HID, NP)),
                  wspec((HID, 1)), wspec((HID, HID)), wspec((HID, 1)),
                  wspec((1, HID)), wspec((1, 1))],
        out_specs=pl.BlockSpec((1, TB), lambda i: (0, i)),
        compiler_params=pltpu.CompilerParams(dimension_semantics=("arbitrary",)),
    )(x_t, *weights)

    return out.reshape(Bp, 1)[:B]


# speedup vs baseline: 1.0347x; 1.0347x over previous
import jax
import jax.numpy as jnp
from jax.experimental import pallas as pl
from jax.experimental.pallas import tpu as pltpu

L_IN = 420
KW = 28
L_OUT = 15
C_OUT = 32
HID = 32
F = L_OUT * C_OUT
P0 = 8
P1 = L_OUT - P0
K0 = P0 * KW
K1 = P1 * KW
N0 = P0 * C_OUT
N1 = P1 * C_OUT
NP = 256


def _net_kernel(x_ref, w0_ref, bb0_ref, w1a_ref,
                w1c_ref, bb1_ref, w1b_ref,
                b1_ref, w2_ref, b2_ref, w3_ref, b3_ref, out_ref):
    xb = x_ref[...]
    zero = jnp.bfloat16(0.0)
    h0 = jnp.dot(w0_ref[...], xb[:K0, :],
                 preferred_element_type=jnp.float32).astype(jnp.bfloat16)
    h0 = jnp.maximum(h0 + bb0_ref[...], zero)
    h1 = jnp.dot(w1c_ref[...], xb[K0:, :],
                 preferred_element_type=jnp.float32).astype(jnp.bfloat16)
    h1 = jnp.maximum(h1 + bb1_ref[...], zero)
    yt = (jnp.dot(w1a_ref[...], h0, preferred_element_type=jnp.float32)
          + jnp.dot(w1b_ref[...], h1, preferred_element_type=jnp.float32))
    yt = jnp.maximum(yt.astype(jnp.bfloat16) + b1_ref[...], zero)
    zt = jnp.dot(w2_ref[...], yt, preferred_element_type=jnp.float32)
    zt = jnp.maximum(zt.astype(jnp.bfloat16) + b2_ref[...], zero)
    logit = jnp.dot(w3_ref[...], zt,
                    preferred_element_type=jnp.float32) + b3_ref[...]
    out_ref[...] = jax.nn.sigmoid(logit)


def _prep_weights(wc, bc, w1, b1, w2, b2, w3, b3):
    wct = jnp.transpose(wc[:, 0, :]).astype(jnp.float32)
    def blockdiag_t(npos):
        eye = jnp.eye(npos, dtype=jnp.float32)
        w = jnp.einsum('lm,kc->lkmc', eye, wct).reshape(npos * KW, npos * C_OUT)
        return jnp.transpose(w)

    w0 = blockdiag_t(P0)
    w1c = jnp.pad(blockdiag_t(P1), ((0, NP - N1), (0, 0)))
    bb0 = jnp.tile(bc, P0).reshape(N0, 1).astype(jnp.bfloat16)
    bb1 = jnp.pad(jnp.tile(bc, P1), (0, NP - N1)).reshape(NP, 1).astype(jnp.bfloat16)
    w1r = jnp.transpose(w1.reshape(HID, C_OUT, L_OUT), (2, 1, 0)).reshape(F, HID)
    w1t = jnp.transpose(w1r)
    w1a = w1t[:, :N0].astype(jnp.bfloat16)
    w1b = jnp.pad(w1t[:, N0:], ((0, 0), (0, NP - N1))).astype(jnp.bfloat16)
    b1c = b1.reshape(HID, 1).astype(jnp.bfloat16)
    w2n = w2.astype(jnp.bfloat16)
    b2c = b2.reshape(HID, 1).astype(jnp.bfloat16)
    w3n = w3.astype(jnp.bfloat16)
    b3r = b3.reshape(1, 1)
    return w0, bb0, w1a, w1c, bb1, w1b, b1c, w2n, b2c, w3n, b3r


@jax.jit
def kernel(x, wc, bc, w1, b1, w2, b2, w3, b3):
    B = x.shape[0]
    weights = _prep_weights(wc, bc, w1, b1, w2, b2, w3, b3)

    x_t = jnp.transpose(x.reshape(B, L_IN))

    TB = min(4096, max(128, ((B + 127) // 128) * 128))
    Bp = ((B + TB - 1) // TB) * TB
    if Bp != B:
        x_t = jnp.pad(x_t, ((0, 0), (0, Bp - B)))
    grid = (Bp // TB,)

    def wspec(shape):
        return pl.BlockSpec(shape, lambda i: (0, 0))

    out = pl.pallas_call(
        _net_kernel,
        out_shape=jax.ShapeDtypeStruct((1, Bp), jnp.float32),
        grid=grid,
        in_specs=[pl.BlockSpec((L_IN, TB), lambda i: (0, i)),
                  wspec((N0, K0)), wspec((N0, 1)), wspec((HID, N0)),
                  wspec((NP, K1)), wspec((NP, 1)), wspec((HID, NP)),
                  wspec((HID, 1)), wspec((HID, HID)), wspec((HID, 1)),
                  wspec((1, HID)), wspec((1, 1))],
        out_specs=pl.BlockSpec((1, TB), lambda i: (0, i)),
        compiler_params=pltpu.CompilerParams(dimension_semantics=("arbitrary",)),
    )(x_t, *weights)

    return out.reshape(Bp, 1)[:B]
